# Initial kernel scaffold; baseline (speedup 1.0000x reference)
import jax
import jax.numpy as jnp
from jax.experimental import pallas as pl


def kernel(atomic_numbers, e0s_tensor):
    raise NotImplementedError("write your pallas kernel here")


# speedup vs baseline: 371.4700x; 371.4700x over previous
import functools

import jax
import jax.numpy as jnp
from jax import lax
from jax.experimental import pallas as pl
from jax.experimental.pallas import tpu as pltpu
from jax.experimental.pallas import tpu_sc as plsc

N = 2097152
TABLE_PAD = 128

_info = plsc.get_sparse_core_info()
_NC, _NS, _L = _info.num_cores, _info.num_subcores, _info.num_lanes
_NW = _NC * _NS
N_PER_W = N // _NW
CHUNK = 8192
N_CHUNKS = N_PER_W // CHUNK


def _make_sc_kernel():
    mesh = plsc.VectorSubcoreMesh(core_axis_name="c", subcore_axis_name="s")

    @functools.partial(
        pl.kernel,
        mesh=mesh,
        out_type=jax.ShapeDtypeStruct((N,), jnp.float32),
        compiler_params=pltpu.CompilerParams(needs_layout_passes=False),
        scratch_types=[
            pltpu.VMEM((TABLE_PAD,), jnp.float32),
            pltpu.VMEM((CHUNK,), jnp.int32),
            pltpu.VMEM((CHUNK,), jnp.float32),
        ],
    )
    def gather_kernel(idx_hbm, table_hbm, out_hbm, table_v, idx_v, out_v):
        wid = lax.axis_index("s") * _NC + lax.axis_index("c")
        base = wid * N_PER_W
        pltpu.sync_copy(table_hbm, table_v)

        def do_chunk(ci, carry):
            off = base + ci * CHUNK
            pltpu.sync_copy(idx_hbm.at[pl.ds(off, CHUNK)], idx_v)

            def body(i, c):
                idx = idx_v[pl.ds(i * _L, _L)]
                out_v[pl.ds(i * _L, _L)] = plsc.load_gather(table_v, [idx])
                return c

            lax.fori_loop(0, CHUNK // _L, body, 0)
            pltpu.sync_copy(out_v, out_hbm.at[pl.ds(off, CHUNK)])
            return carry

        lax.fori_loop(0, N_CHUNKS, do_chunk, 0)

    return gather_kernel


_sc_kernel = _make_sc_kernel()


def kernel(atomic_numbers, e0s_tensor):
    idx = atomic_numbers.astype(jnp.int32)
    table = jnp.zeros((TABLE_PAD,), jnp.float32).at[: e0s_tensor.shape[0]].set(
        e0s_tensor
    )
    return _sc_kernel(idx, table)


# speedup vs baseline: 662.3225x; 1.7830x over previous
import functools

import jax
import jax.numpy as jnp
from jax import lax
from jax.experimental import pallas as pl
from jax.experimental.pallas import tpu as pltpu
from jax.experimental.pallas import tpu_sc as plsc

N = 2097152
TABLE_PAD = 128

_info = plsc.get_sparse_core_info()
_NC, _NS, _L = _info.num_cores, _info.num_subcores, _info.num_lanes
_NW = _NC * _NS
N_PER_W = N // _NW
CHUNK = 16384
N_CHUNKS = N_PER_W // CHUNK
UNROLL = 8


def _make_sc_kernel():
    mesh = plsc.VectorSubcoreMesh(core_axis_name="c", subcore_axis_name="s")

    @functools.partial(
        pl.kernel,
        mesh=mesh,
        out_type=jax.ShapeDtypeStruct((N,), jnp.float32),
        compiler_params=pltpu.CompilerParams(needs_layout_passes=False),
        scratch_types=[
            pltpu.VMEM((TABLE_PAD,), jnp.float32),
            pltpu.VMEM((2, CHUNK), jnp.int32),
            pltpu.VMEM((2, CHUNK), jnp.float32),
            pltpu.SemaphoreType.DMA,
            pltpu.SemaphoreType.DMA,
            pltpu.SemaphoreType.DMA,
            pltpu.SemaphoreType.DMA,
        ],
    )
    def gather_kernel(
        idx_hbm, table_hbm, out_hbm, table_v, idx_v, out_v, is0, is1, os0, os1
    ):
        wid = lax.axis_index("s") * _NC + lax.axis_index("c")
        base = wid * N_PER_W
        pltpu.sync_copy(table_hbm, table_v)
        isems = (is0, is1)
        osems = (os0, os1)
        in_h = [None, None]
        out_h = [None, None]
        in_h[0] = pltpu.async_copy(
            idx_hbm.at[pl.ds(base, CHUNK)], idx_v.at[0], isems[0]
        )
        for ci in range(N_CHUNKS):
            b = ci % 2
            nb = 1 - b
            if ci + 1 < N_CHUNKS:
                in_h[nb] = pltpu.async_copy(
                    idx_hbm.at[pl.ds(base + (ci + 1) * CHUNK, CHUNK)],
                    idx_v.at[nb],
                    isems[nb],
                )
            in_h[b].wait()
            if out_h[b] is not None:
                out_h[b].wait()

            @plsc.parallel_loop(0, CHUNK // _L, unroll=UNROLL)
            def _body(i, _b=b):
                o = i * _L
                idx = idx_v[_b, pl.ds(o, _L)]
                out_v[_b, pl.ds(o, _L)] = plsc.load_gather(table_v, [idx])

            out_h[b] = pltpu.async_copy(
                out_v.at[b], out_hbm.at[pl.ds(base + ci * CHUNK, CHUNK)], osems[b]
            )
        out_h[0].wait()
        out_h[1].wait()

    return gather_kernel


_sc_kernel = _make_sc_kernel()


def kernel(atomic_numbers, e0s_tensor):
    idx = atomic_numbers.astype(jnp.int32)
    table = jnp.zeros((TABLE_PAD,), jnp.float32).at[: e0s_tensor.shape[0]].set(
        e0s_tensor
    )
    return _sc_kernel(idx, table)


# speedup vs baseline: 667.8687x; 1.0084x over previous
import functools

import jax
import jax.numpy as jnp
from jax import lax
from jax.experimental import pallas as pl
from jax.experimental.pallas import tpu as pltpu
from jax.experimental.pallas import tpu_sc as plsc

N = 2097152
TABLE_SIZE = 123
TABLE_PAD = 128

_info = plsc.get_sparse_core_info()
_NC, _NS, _L = _info.num_cores, _info.num_subcores, _info.num_lanes
_NW = _NC * _NS
N_PER_W = N // _NW
CHUNK = 16384
N_CHUNKS = N_PER_W // CHUNK
UNROLL = 8


def _make_sc_kernel():
    mesh = plsc.VectorSubcoreMesh(core_axis_name="c", subcore_axis_name="s")

    @functools.partial(
        pl.kernel,
        mesh=mesh,
        out_type=jax.ShapeDtypeStruct((N,), jnp.float32),
        compiler_params=pltpu.CompilerParams(needs_layout_passes=False),
        scratch_types=[
            pltpu.VMEM((TABLE_PAD,), jnp.float32),
            pltpu.VMEM((2, CHUNK), jnp.int32),
            pltpu.VMEM((2, CHUNK), jnp.float32),
            pltpu.SemaphoreType.DMA,
            pltpu.SemaphoreType.DMA,
            pltpu.SemaphoreType.DMA,
            pltpu.SemaphoreType.DMA,
        ],
    )
    def gather_kernel(
        idx_hbm, table_hbm, out_hbm, table_v, idx_v, out_v, is0, is1, os0, os1
    ):
        wid = lax.axis_index("s") * _NC + lax.axis_index("c")
        base = wid * N_PER_W
        pltpu.sync_copy(table_hbm, table_v.at[pl.ds(0, TABLE_SIZE)])
        isems = (is0, is1)
        osems = (os0, os1)
        in_h = [None, None]
        out_h = [None, None]
        in_h[0] = pltpu.async_copy(
            idx_hbm.at[pl.ds(base, CHUNK)], idx_v.at[0], isems[0]
        )
        for ci in range(N_CHUNKS):
            b = ci % 2
            nb = 1 - b
            if ci + 1 < N_CHUNKS:
                in_h[nb] = pltpu.async_copy(
                    idx_hbm.at[pl.ds(base + (ci + 1) * CHUNK, CHUNK)],
                    idx_v.at[nb],
                    isems[nb],
                )
            in_h[b].wait()
            if out_h[b] is not None:
                out_h[b].wait()

            @plsc.parallel_loop(0, CHUNK // _L, unroll=UNROLL)
            def _body(i, _b=b):
                o = i * _L
                idx = idx_v[_b, pl.ds(o, _L)]
                out_v[_b, pl.ds(o, _L)] = plsc.load_gather(table_v, [idx])

            out_h[b] = pltpu.async_copy(
                out_v.at[b], out_hbm.at[pl.ds(base + ci * CHUNK, CHUNK)], osems[b]
            )
        out_h[0].wait()
        out_h[1].wait()

    return gather_kernel


_sc_kernel = _make_sc_kernel()


def kernel(atomic_numbers, e0s_tensor):
    return _sc_kernel(atomic_numbers.astype(jnp.int32), e0s_tensor)
